# baseline (speedup 1.0000x reference)
import functools

import jax
import jax.numpy as jnp
import numpy as np
from jax import lax
from jax.experimental import pallas as pl
from jax.experimental.pallas import tpu as pltpu

_pcall = pl.pallas_call
_PHI = lax.Precision.HIGHEST

F32 = jnp.float32


def _sc_gather_impl(table, idx):
    from jax.experimental.pallas import tpu_sc as plsc

    info = plsc.get_sparse_core_info()
    nc, ns = info.num_cores, info.num_subcores
    nw = nc * ns
    b = idx.shape[0]
    d = table.shape[1]
    bpw = b // nw
    mesh = plsc.VectorSubcoreMesh(core_axis_name="c", subcore_axis_name="s")

    @functools.partial(
        pl.kernel,
        mesh=mesh,
        compiler_params=pltpu.CompilerParams(use_tc_tiling_on_sc=False),
        out_type=jax.ShapeDtypeStruct((b, d), F32),
        scratch_types=[
            pltpu.VMEM((bpw,), jnp.int32),
            pltpu.VMEM((bpw, d), F32),
            pltpu.SemaphoreType.DMA,
        ],
    )
    def gk(table_hbm, idx_hbm, out_hbm, idx_v, rows_v, sem):
        wid = lax.axis_index("s") * nc + lax.axis_index("c")
        base = wid * bpw
        pltpu.sync_copy(idx_hbm.at[pl.ds(base, bpw)], idx_v)
        pltpu.async_copy(table_hbm.at[idx_v], rows_v, sem).wait()
        pltpu.sync_copy(rows_v, out_hbm.at[pl.ds(base, bpw)])

    return gk(table, idx)


def _char_kernel(ids_ref, t_ref, w_ref, b_ref, o_ref):
    n = ids_ref.shape[0]
    m = jnp.dot(t_ref[...], w_ref[...], preferred_element_type=F32, precision=_PHI)
    ids = ids_ref[...]
    iot = lax.broadcasted_iota(jnp.int32, (n, 128), 1)
    zs = []
    for c in range(14):
        oh = (ids[:, c : c + 1] == iot).astype(F32)
        zs.append(jnp.dot(oh, m, preferred_element_type=F32, precision=_PHI))
    best = None
    for pos in range(10):
        y = zs[pos][:, 0:128]
        for k in range(1, 5):
            y = y + zs[pos + k][:, 128 * k : 128 * k + 128]
        best = y if best is None else jnp.maximum(best, y)
    o_ref[...] = best[:, :77] + b_ref[...]


def _char_call(cids, ctable, w2p, cbias):
    nw = cids.shape[0]
    blk = 256
    return _pcall(
        _char_kernel,
        grid=(nw // blk,),
        in_specs=[
            pl.BlockSpec((blk, 14), lambda i: (i, 0)),
            pl.BlockSpec((128, 30), lambda i: (0, 0)),
            pl.BlockSpec((30, 640), lambda i: (0, 0)),
            pl.BlockSpec((1, 77), lambda i: (0, 0)),
        ],
        out_specs=pl.BlockSpec((blk, 77), lambda i: (i, 0)),
        out_shape=jax.ShapeDtypeStruct((nw, 77), F32),
    )(cids, ctable, w2p, cbias)


def _enc_kernel(emb_ref, ws_ref, bs_ref, wi_ref, o_ref):
    P = emb_ref[0]
    wa = wi_ref[0, 0:1, :]
    wb = wi_ref[0, 1:2, :]
    wc = wi_ref[0, 2:3, :]
    pa = jnp.sum(P * wa, axis=1, keepdims=True)
    dn = (((1,), (1,)), ((), ()))
    pbt = lax.dot_general(wb, P, dn, preferred_element_type=F32, precision=_PHI)
    cc = lax.dot_general(P * wc, P, dn, preferred_element_type=F32, precision=_PHI)
    a = pa + pbt + cc
    a = a - jnp.max(a, axis=1, keepdims=True)
    e = jnp.exp(a)
    att = e / jnp.sum(e, axis=1, keepdims=True)
    itr = jnp.dot(att, P, preferred_element_type=F32, precision=_PHI)
    cat = jnp.concatenate([P, itr], axis=1)
    ws = ws_ref[0]
    z = jnp.tanh(jnp.dot(cat, ws[0:854], preferred_element_type=F32, precision=_PHI) + bs_ref[0, 0:1, :])
    r = jax.nn.sigmoid(jnp.dot(cat, ws[854:1708], preferred_element_type=F32, precision=_PHI) + bs_ref[0, 1:2, :])
    f = jax.nn.sigmoid(jnp.dot(cat, ws[1708:2562], preferred_element_type=F32, precision=_PHI) + bs_ref[0, 2:3, :])
    o_ref[0] = r * P + f * z


def _enc_call(emb, ws, bs, wi):
    return _pcall(
        _enc_kernel,
        grid=(32,),
        in_specs=[
            pl.BlockSpec((1, 48, 427), lambda i: (i, 0, 0)),
            pl.BlockSpec((1, 2562, 427), lambda i: (i // 16, 0, 0)),
            pl.BlockSpec((1, 3, 427), lambda i: (i // 16, 0, 0)),
            pl.BlockSpec((1, 3, 427), lambda i: (i // 16, 0, 0)),
        ],
        out_specs=pl.BlockSpec((1, 48, 427), lambda i: (i, 0, 0)),
        out_shape=jax.ShapeDtypeStruct((32, 48, 427), F32),
    )(emb, ws, bs, wi)


def _inter_kernel(p_ref, h_ref, w_ref, b_ref, o_ref):
    p = p_ref[0]
    h = h_ref[0]
    p3 = jnp.broadcast_to(p[:, None, :], (48, 48, 427)).reshape(2304, 427)
    h3 = jnp.broadcast_to(h[None, :, :], (48, 48, 427)).reshape(2304, 427)
    inter = p3 * h3
    o_ref[0] = jnp.dot(inter, w_ref[...], preferred_element_type=F32, precision=_PHI) + b_ref[...]


def _inter_call(penc, henc, w, b):
    return _pcall(
        _inter_kernel,
        grid=(16,),
        in_specs=[
            pl.BlockSpec((1, 48, 427), lambda i: (i, 0, 0)),
            pl.BlockSpec((1, 48, 427), lambda i: (i, 0, 0)),
            pl.BlockSpec((427, 128), lambda i: (0, 0)),
            pl.BlockSpec((1, 128), lambda i: (0, 0)),
        ],
        out_specs=pl.BlockSpec((1, 2304, 128), lambda i: (i, 0, 0)),
        out_shape=jax.ShapeDtypeStruct((16, 2304, 128), F32),
    )(penc, henc, w, b)


def _sel_np(g):
    h = g // 2
    s = np.zeros((h * h, g * g), np.float32)
    r = np.arange(h * h)
    s[r, (2 * (r // h)) * g + 2 * (r % h)] = 1.0
    return s


_SEL = {g: _sel_np(g) for g in (48, 24, 12)}


def _block_kernel(x_ref, wc_ref, wt_ref, s_ref, o_ref, x_s, *, g, c0):
    gg = g * g
    x_s[:, :c0] = x_ref[0]
    jm = lax.broadcasted_iota(jnp.int32, (gg, 20), 0) % g
    mask_l = jm >= 1
    mask_r = jm <= g - 2
    zpad = jnp.zeros((g + 1, 180), F32)
    c = c0
    off = 0
    for _l in range(8):
        xv = x_s[:, :c]
        z = jnp.dot(xv, wc_ref[off : off + c, :], preferred_element_type=F32, precision=_PHI)
        zp = jnp.concatenate([zpad, z, zpad], axis=0)
        acc = jnp.zeros((gg, 20), F32)
        t = 0
        for a in (0, 1, 2):
            for b in (0, 1, 2):
                sh = (a - 1) * g + (b - 1)
                sl = zp[(g + 1) + sh : (g + 1) + sh + gg, 20 * t : 20 * t + 20]
                if b == 0:
                    sl = jnp.where(mask_l, sl, 0.0)
                elif b == 2:
                    sl = jnp.where(mask_r, sl, 0.0)
                acc = acc + sl
                t += 1
        x_s[:, c : c + 20] = jnp.maximum(acc, 0.0)
        off += c
        c += 20
    tr = jnp.maximum(jnp.dot(x_s[...], wt_ref[...], preferred_element_type=F32, precision=_PHI), 0.0)
    co = tr.shape[1]
    sh1 = jnp.concatenate([tr[1:], jnp.zeros((1, co), F32)], axis=0)
    m1 = jnp.maximum(tr, sh1)
    shg = jnp.concatenate([m1[g:], jnp.zeros((g, co), F32)], axis=0)
    m2 = jnp.maximum(m1, shg)
    o_ref[0] = jnp.dot(s_ref[...], m2, preferred_element_type=F32, precision=_PHI)


def _block_call(x, dn, b, g, c0):
    gg = g * g
    cfin = c0 + 160
    co = cfin // 2
    wcat = jnp.concatenate(
        [
            jnp.transpose(dn["b%d_l%d" % (b, l)], (2, 0, 1, 3)).reshape(c0 + 20 * l, 180)
            for l in range(8)
        ],
        axis=0,
    )
    sumc = wcat.shape[0]
    wt = dn["t%d" % b].reshape(cfin, co)
    sel = jnp.asarray(_SEL[g])
    return _pcall(
        functools.partial(_block_kernel, g=g, c0=c0),
        grid=(16,),
        in_specs=[
            pl.BlockSpec((1, gg, c0), lambda i: (i, 0, 0)),
            pl.BlockSpec((sumc, 180), lambda i: (0, 0)),
            pl.BlockSpec((cfin, co), lambda i: (0, 0)),
            pl.BlockSpec((gg // 4, gg), lambda i: (0, 0)),
        ],
        out_specs=pl.BlockSpec((1, gg // 4, co), lambda i: (i, 0, 0)),
        out_shape=jax.ShapeDtypeStruct((16, gg // 4, co), F32),
        scratch_shapes=[pltpu.VMEM((gg, cfin), F32)],
    )(x, wcat, wt, sel)


def _final_kernel(x_ref, w_ref, b_ref, o_ref):
    logits = jnp.dot(x_ref[...], w_ref[...], preferred_element_type=F32, precision=_PHI) + b_ref[...]
    logits = logits - jnp.max(logits, axis=1, keepdims=True)
    e = jnp.exp(logits)
    o_ref[...] = e / jnp.sum(e, axis=1, keepdims=True)


def _final_call(x, w, b):
    k = x.shape[1]
    return _pcall(
        _final_kernel,
        grid=(1,),
        in_specs=[
            pl.BlockSpec((16, k), lambda i: (0, 0)),
            pl.BlockSpec((k, 3), lambda i: (0, 0)),
            pl.BlockSpec((1, 3), lambda i: (0, 0)),
        ],
        out_specs=pl.BlockSpec((16, 3), lambda i: (0, 0)),
        out_shape=jax.ShapeDtypeStruct((16, 3), F32),
    )(x, w, b)


def kernel(prem_word, prem_char, prem_syn, hyp_word, hyp_char, hyp_syn, params):
    p = params
    idx = jnp.concatenate(
        [prem_word.reshape(-1), hyp_word.reshape(-1)]
    ).astype(jnp.int32)
    table = jnp.pad(p["word_table"], ((0, 0), (0, 4)))
    wemb = _sc_gather_impl(table, idx)[:, :300]

    cids = jnp.concatenate(
        [prem_char.reshape(-1, 14), hyp_char.reshape(-1, 14)]
    ).astype(jnp.int32)
    wtr = jnp.transpose(p["char_conv_w"], (1, 0, 2))
    w2p = jnp.pad(wtr, ((0, 0), (0, 0), (0, 51))).reshape(30, 640)
    chf = _char_call(cids, p["char_table"], w2p, p["char_conv_b"].reshape(1, 77))

    syn = jnp.concatenate(
        [prem_syn.reshape(-1, 50), hyp_syn.reshape(-1, 50)]
    ).astype(F32)
    emb = jnp.concatenate([wemb, chf, syn], axis=1).reshape(32, 48, 427)

    encs = (p["enc_p"], p["enc_h"])
    ws = jnp.stack([jnp.concatenate([e["W1"], e["W2"], e["W3"]], axis=0) for e in encs])
    bs = jnp.stack([jnp.stack([e["b1"], e["b2"], e["b3"]]) for e in encs])
    wi = jnp.stack([e["w_itr"].reshape(3, 427) for e in encs])
    enc = _enc_call(emb, ws, bs, wi)

    x = _inter_call(
        enc[:16], enc[16:], p["fsd_w"].reshape(427, 128), p["fsd_b"].reshape(1, 128)
    )

    dn = p["densenet"]
    x = _block_call(x, dn, 0, 48, 128)
    x = _block_call(x, dn, 1, 24, 144)
    x = _block_call(x, dn, 2, 12, 152)

    xf = x.reshape(16, 36 * 156)
    return _final_call(xf, p["out_w"], p["out_b"].reshape(1, 3))


# speedup vs baseline: 1.7792x; 1.7792x over previous
import functools

import jax
import jax.numpy as jnp
import numpy as np
from jax import lax
from jax.experimental import pallas as pl
from jax.experimental.pallas import tpu as pltpu

_pcall = pl.pallas_call
_PHI = lax.Precision.DEFAULT

F32 = jnp.float32


def _sc_gather_impl(table, idx):
    from jax.experimental.pallas import tpu_sc as plsc

    info = plsc.get_sparse_core_info()
    nc, ns = info.num_cores, info.num_subcores
    nw = nc * ns
    b = idx.shape[0]
    d = table.shape[1]
    bpw = b // nw
    mesh = plsc.VectorSubcoreMesh(core_axis_name="c", subcore_axis_name="s")

    @functools.partial(
        pl.kernel,
        mesh=mesh,
        compiler_params=pltpu.CompilerParams(use_tc_tiling_on_sc=False),
        out_type=jax.ShapeDtypeStruct((b, d), F32),
        scratch_types=[
            pltpu.VMEM((bpw,), jnp.int32),
            pltpu.VMEM((bpw, d), F32),
            pltpu.SemaphoreType.DMA,
        ],
    )
    def gk(table_hbm, idx_hbm, out_hbm, idx_v, rows_v, sem):
        wid = lax.axis_index("s") * nc + lax.axis_index("c")
        base = wid * bpw
        pltpu.sync_copy(idx_hbm.at[pl.ds(base, bpw)], idx_v)
        pltpu.async_copy(table_hbm.at[idx_v], rows_v, sem).wait()
        pltpu.sync_copy(rows_v, out_hbm.at[pl.ds(base, bpw)])

    return gk(table, idx)


def _char_kernel(ids_ref, t_ref, w_ref, b_ref, o_ref):
    n = ids_ref.shape[0]
    m = jnp.dot(t_ref[...], w_ref[...], preferred_element_type=F32, precision=_PHI)
    ids = ids_ref[...]
    iot = lax.broadcasted_iota(jnp.int32, (n, 128), 1)
    zs = []
    for c in range(14):
        oh = (ids[:, c : c + 1] == iot).astype(F32)
        zs.append(jnp.dot(oh, m, preferred_element_type=F32, precision=_PHI))
    best = None
    for pos in range(10):
        y = zs[pos][:, 0:128]
        for k in range(1, 5):
            y = y + zs[pos + k][:, 128 * k : 128 * k + 128]
        best = y if best is None else jnp.maximum(best, y)
    o_ref[...] = best[:, :77] + b_ref[...]


def _char_call(cids, ctable, w2p, cbias):
    nw = cids.shape[0]
    blk = 256
    return _pcall(
        _char_kernel,
        grid=(nw // blk,),
        in_specs=[
            pl.BlockSpec((blk, 14), lambda i: (i, 0)),
            pl.BlockSpec((128, 30), lambda i: (0, 0)),
            pl.BlockSpec((30, 640), lambda i: (0, 0)),
            pl.BlockSpec((1, 77), lambda i: (0, 0)),
        ],
        out_specs=pl.BlockSpec((blk, 77), lambda i: (i, 0)),
        out_shape=jax.ShapeDtypeStruct((nw, 77), F32),
    )(cids, ctable, w2p, cbias)


def _enc_kernel(emb_ref, ws_ref, bs_ref, wi_ref, o_ref):
    P = emb_ref[0]
    wa = wi_ref[0, 0:1, :]
    wb = wi_ref[0, 1:2, :]
    wc = wi_ref[0, 2:3, :]
    pa = jnp.sum(P * wa, axis=1, keepdims=True)
    dn = (((1,), (1,)), ((), ()))
    pbt = lax.dot_general(wb, P, dn, preferred_element_type=F32, precision=_PHI)
    cc = lax.dot_general(P * wc, P, dn, preferred_element_type=F32, precision=_PHI)
    a = pa + pbt + cc
    a = a - jnp.max(a, axis=1, keepdims=True)
    e = jnp.exp(a)
    att = e / jnp.sum(e, axis=1, keepdims=True)
    itr = jnp.dot(att, P, preferred_element_type=F32, precision=_PHI)
    cat = jnp.concatenate([P, itr], axis=1)
    ws = ws_ref[0]
    z = jnp.tanh(jnp.dot(cat, ws[0:854], preferred_element_type=F32, precision=_PHI) + bs_ref[0, 0:1, :])
    r = jax.nn.sigmoid(jnp.dot(cat, ws[854:1708], preferred_element_type=F32, precision=_PHI) + bs_ref[0, 1:2, :])
    f = jax.nn.sigmoid(jnp.dot(cat, ws[1708:2562], preferred_element_type=F32, precision=_PHI) + bs_ref[0, 2:3, :])
    o_ref[0] = r * P + f * z


def _enc_call(emb, ws, bs, wi):
    return _pcall(
        _enc_kernel,
        grid=(32,),
        in_specs=[
            pl.BlockSpec((1, 48, 427), lambda i: (i, 0, 0)),
            pl.BlockSpec((1, 2562, 427), lambda i: (i // 16, 0, 0)),
            pl.BlockSpec((1, 3, 427), lambda i: (i // 16, 0, 0)),
            pl.BlockSpec((1, 3, 427), lambda i: (i // 16, 0, 0)),
        ],
        out_specs=pl.BlockSpec((1, 48, 427), lambda i: (i, 0, 0)),
        out_shape=jax.ShapeDtypeStruct((32, 48, 427), F32),
    )(emb, ws, bs, wi)


def _inter_kernel(p_ref, h_ref, w_ref, b_ref, o_ref):
    p = p_ref[0]
    h = h_ref[0]
    p3 = jnp.broadcast_to(p[:, None, :], (48, 48, 427)).reshape(2304, 427)
    h3 = jnp.broadcast_to(h[None, :, :], (48, 48, 427)).reshape(2304, 427)
    inter = p3 * h3
    o_ref[0] = jnp.dot(inter, w_ref[...], preferred_element_type=F32, precision=_PHI) + b_ref[...]


def _inter_call(penc, henc, w, b):
    return _pcall(
        _inter_kernel,
        grid=(16,),
        in_specs=[
            pl.BlockSpec((1, 48, 427), lambda i: (i, 0, 0)),
            pl.BlockSpec((1, 48, 427), lambda i: (i, 0, 0)),
            pl.BlockSpec((427, 128), lambda i: (0, 0)),
            pl.BlockSpec((1, 128), lambda i: (0, 0)),
        ],
        out_specs=pl.BlockSpec((1, 2304, 128), lambda i: (i, 0, 0)),
        out_shape=jax.ShapeDtypeStruct((16, 2304, 128), F32),
    )(penc, henc, w, b)


def _sel_np(g):
    h = g // 2
    s = np.zeros((h * h, g * g), np.float32)
    r = np.arange(h * h)
    s[r, (2 * (r // h)) * g + 2 * (r % h)] = 1.0
    return s


_SEL = {g: _sel_np(g) for g in (48, 24, 12)}


def _block_kernel(x_ref, wc_ref, wt_ref, s_ref, o_ref, x_s, *, g, c0):
    gg = g * g
    x_s[:, :c0] = x_ref[0]
    jm = lax.broadcasted_iota(jnp.int32, (gg, 20), 0) % g
    mask_l = jm >= 1
    mask_r = jm <= g - 2
    zpad = jnp.zeros((g + 1, 180), F32)
    c = c0
    off = 0
    for _l in range(8):
        xv = x_s[:, :c]
        z = jnp.dot(xv, wc_ref[off : off + c, :], preferred_element_type=F32, precision=_PHI)
        zp = jnp.concatenate([zpad, z, zpad], axis=0)
        acc = jnp.zeros((gg, 20), F32)
        t = 0
        for a in (0, 1, 2):
            for b in (0, 1, 2):
                sh = (a - 1) * g + (b - 1)
                sl = zp[(g + 1) + sh : (g + 1) + sh + gg, 20 * t : 20 * t + 20]
                if b == 0:
                    sl = jnp.where(mask_l, sl, 0.0)
                elif b == 2:
                    sl = jnp.where(mask_r, sl, 0.0)
                acc = acc + sl
                t += 1
        x_s[:, c : c + 20] = jnp.maximum(acc, 0.0)
        off += c
        c += 20
    tr = jnp.maximum(jnp.dot(x_s[...], wt_ref[...], preferred_element_type=F32, precision=_PHI), 0.0)
    co = tr.shape[1]
    sh1 = jnp.concatenate([tr[1:], jnp.zeros((1, co), F32)], axis=0)
    m1 = jnp.maximum(tr, sh1)
    shg = jnp.concatenate([m1[g:], jnp.zeros((g, co), F32)], axis=0)
    m2 = jnp.maximum(m1, shg)
    o_ref[0] = jnp.dot(s_ref[...], m2, preferred_element_type=F32, precision=_PHI)


def _block_call(x, dn, b, g, c0):
    gg = g * g
    cfin = c0 + 160
    co = cfin // 2
    wcat = jnp.concatenate(
        [
            jnp.transpose(dn["b%d_l%d" % (b, l)], (2, 0, 1, 3)).reshape(c0 + 20 * l, 180)
            for l in range(8)
        ],
        axis=0,
    )
    sumc = wcat.shape[0]
    wt = dn["t%d" % b].reshape(cfin, co)
    sel = jnp.asarray(_SEL[g])
    return _pcall(
        functools.partial(_block_kernel, g=g, c0=c0),
        grid=(16,),
        in_specs=[
            pl.BlockSpec((1, gg, c0), lambda i: (i, 0, 0)),
            pl.BlockSpec((sumc, 180), lambda i: (0, 0)),
            pl.BlockSpec((cfin, co), lambda i: (0, 0)),
            pl.BlockSpec((gg // 4, gg), lambda i: (0, 0)),
        ],
        out_specs=pl.BlockSpec((1, gg // 4, co), lambda i: (i, 0, 0)),
        out_shape=jax.ShapeDtypeStruct((16, gg // 4, co), F32),
        scratch_shapes=[pltpu.VMEM((gg, cfin), F32)],
    )(x, wcat, wt, sel)


def _final_kernel(x_ref, w_ref, b_ref, o_ref):
    logits = jnp.dot(x_ref[...], w_ref[...], preferred_element_type=F32, precision=_PHI) + b_ref[...]
    logits = logits - jnp.max(logits, axis=1, keepdims=True)
    e = jnp.exp(logits)
    o_ref[...] = e / jnp.sum(e, axis=1, keepdims=True)


def _final_call(x, w, b):
    k = x.shape[1]
    return _pcall(
        _final_kernel,
        grid=(1,),
        in_specs=[
            pl.BlockSpec((16, k), lambda i: (0, 0)),
            pl.BlockSpec((k, 3), lambda i: (0, 0)),
            pl.BlockSpec((1, 3), lambda i: (0, 0)),
        ],
        out_specs=pl.BlockSpec((16, 3), lambda i: (0, 0)),
        out_shape=jax.ShapeDtypeStruct((16, 3), F32),
    )(x, w, b)


def kernel(prem_word, prem_char, prem_syn, hyp_word, hyp_char, hyp_syn, params):
    p = params
    idx = jnp.concatenate(
        [prem_word.reshape(-1), hyp_word.reshape(-1)]
    ).astype(jnp.int32)
    table = jnp.pad(p["word_table"], ((0, 0), (0, 4)))
    wemb = _sc_gather_impl(table, idx)[:, :300]

    cids = jnp.concatenate(
        [prem_char.reshape(-1, 14), hyp_char.reshape(-1, 14)]
    ).astype(jnp.int32)
    wtr = jnp.transpose(p["char_conv_w"], (1, 0, 2))
    w2p = jnp.pad(wtr, ((0, 0), (0, 0), (0, 51))).reshape(30, 640)
    chf = _char_call(cids, p["char_table"], w2p, p["char_conv_b"].reshape(1, 77))

    syn = jnp.concatenate(
        [prem_syn.reshape(-1, 50), hyp_syn.reshape(-1, 50)]
    ).astype(F32)
    emb = jnp.concatenate([wemb, chf, syn], axis=1).reshape(32, 48, 427)

    encs = (p["enc_p"], p["enc_h"])
    ws = jnp.stack([jnp.concatenate([e["W1"], e["W2"], e["W3"]], axis=0) for e in encs])
    bs = jnp.stack([jnp.stack([e["b1"], e["b2"], e["b3"]]) for e in encs])
    wi = jnp.stack([e["w_itr"].reshape(3, 427) for e in encs])
    enc = _enc_call(emb, ws, bs, wi)

    x = _inter_call(
        enc[:16], enc[16:], p["fsd_w"].reshape(427, 128), p["fsd_b"].reshape(1, 128)
    )

    dn = p["densenet"]
    x = _block_call(x, dn, 0, 48, 128)
    x = _block_call(x, dn, 1, 24, 144)
    x = _block_call(x, dn, 2, 12, 152)

    xf = x.reshape(16, 36 * 156)
    return _final_call(xf, p["out_w"], p["out_b"].reshape(1, 3))


# speedup vs baseline: 1.7975x; 1.0103x over previous
import functools

import jax
import jax.numpy as jnp
import numpy as np
from jax import lax
from jax.experimental import pallas as pl
from jax.experimental.pallas import tpu as pltpu

_pcall = pl.pallas_call
_PHI = lax.Precision.DEFAULT

F32 = jnp.float32


def _sc_gather_impl(table, idx):
    from jax.experimental.pallas import tpu_sc as plsc

    info = plsc.get_sparse_core_info()
    nc, ns = info.num_cores, info.num_subcores
    nw = nc * ns
    b = idx.shape[0]
    d = table.shape[1]
    bpw = b // nw
    mesh = plsc.VectorSubcoreMesh(core_axis_name="c", subcore_axis_name="s")

    @functools.partial(
        pl.kernel,
        mesh=mesh,
        compiler_params=pltpu.CompilerParams(use_tc_tiling_on_sc=False),
        out_type=jax.ShapeDtypeStruct((b, d), F32),
        scratch_types=[
            pltpu.VMEM((bpw,), jnp.int32),
            pltpu.VMEM((bpw, d), F32),
            pltpu.SemaphoreType.DMA,
        ],
    )
    def gk(table_hbm, idx_hbm, out_hbm, idx_v, rows_v, sem):
        wid = lax.axis_index("s") * nc + lax.axis_index("c")
        base = wid * bpw
        pltpu.sync_copy(idx_hbm.at[pl.ds(base, bpw)], idx_v)
        pltpu.async_copy(table_hbm.at[idx_v], rows_v, sem).wait()
        pltpu.sync_copy(rows_v, out_hbm.at[pl.ds(base, bpw)])

    return gk(table, idx)


def _qsel_kernel(x_ref, k_ref, o_ref):
    x = x_ref[...]
    k = k_ref[...]
    s = [x[:, 300 * q : 300 * q + 300] for q in range(4)]
    o_ref[...] = jnp.where(
        k == 0, s[0], jnp.where(k == 1, s[1], jnp.where(k == 2, s[2], s[3]))
    )


def _qsel_call(rows4, kvec):
    n = rows4.shape[0]
    blk = 256
    return _pcall(
        _qsel_kernel,
        grid=(n // blk,),
        in_specs=[
            pl.BlockSpec((blk, 1200), lambda i: (i, 0)),
            pl.BlockSpec((blk, 1), lambda i: (i, 0)),
        ],
        out_specs=pl.BlockSpec((blk, 300), lambda i: (i, 0)),
        out_shape=jax.ShapeDtypeStruct((n, 300), F32),
    )(rows4, kvec)


def _char_kernel(ids_ref, t_ref, w_ref, b_ref, o_ref):
    n = ids_ref.shape[0]
    m = jnp.dot(t_ref[...], w_ref[...], preferred_element_type=F32, precision=_PHI)
    ids = ids_ref[...]
    iot = lax.broadcasted_iota(jnp.int32, (n, 128), 1)
    zs = []
    for c in range(14):
        oh = (ids[:, c : c + 1] == iot).astype(F32)
        zs.append(jnp.dot(oh, m, preferred_element_type=F32, precision=_PHI))
    best = None
    for pos in range(10):
        y = zs[pos][:, 0:128]
        for k in range(1, 5):
            y = y + zs[pos + k][:, 128 * k : 128 * k + 128]
        best = y if best is None else jnp.maximum(best, y)
    o_ref[...] = best[:, :77] + b_ref[...]


def _char_call(cids, ctable, w2p, cbias):
    nw = cids.shape[0]
    blk = 256
    return _pcall(
        _char_kernel,
        grid=(nw // blk,),
        in_specs=[
            pl.BlockSpec((blk, 14), lambda i: (i, 0)),
            pl.BlockSpec((128, 30), lambda i: (0, 0)),
            pl.BlockSpec((30, 640), lambda i: (0, 0)),
            pl.BlockSpec((1, 77), lambda i: (0, 0)),
        ],
        out_specs=pl.BlockSpec((blk, 77), lambda i: (i, 0)),
        out_shape=jax.ShapeDtypeStruct((nw, 77), F32),
    )(cids, ctable, w2p, cbias)


def _enc_kernel(emb_ref, ws_ref, bs_ref, wi_ref, o_ref):
    P = emb_ref[0]
    wa = wi_ref[0, 0:1, :]
    wb = wi_ref[0, 1:2, :]
    wc = wi_ref[0, 2:3, :]
    pa = jnp.sum(P * wa, axis=1, keepdims=True)
    dn = (((1,), (1,)), ((), ()))
    pbt = lax.dot_general(wb, P, dn, preferred_element_type=F32, precision=_PHI)
    cc = lax.dot_general(P * wc, P, dn, preferred_element_type=F32, precision=_PHI)
    a = pa + pbt + cc
    a = a - jnp.max(a, axis=1, keepdims=True)
    e = jnp.exp(a)
    att = e / jnp.sum(e, axis=1, keepdims=True)
    itr = jnp.dot(att, P, preferred_element_type=F32, precision=_PHI)
    cat = jnp.concatenate([P, itr], axis=1)
    ws = ws_ref[0]
    z = jnp.tanh(jnp.dot(cat, ws[0:854], preferred_element_type=F32, precision=_PHI) + bs_ref[0, 0:1, :])
    r = jax.nn.sigmoid(jnp.dot(cat, ws[854:1708], preferred_element_type=F32, precision=_PHI) + bs_ref[0, 1:2, :])
    f = jax.nn.sigmoid(jnp.dot(cat, ws[1708:2562], preferred_element_type=F32, precision=_PHI) + bs_ref[0, 2:3, :])
    o_ref[0] = r * P + f * z


def _enc_call(emb, ws, bs, wi):
    return _pcall(
        _enc_kernel,
        grid=(32,),
        in_specs=[
            pl.BlockSpec((1, 48, 427), lambda i: (i, 0, 0)),
            pl.BlockSpec((1, 2562, 427), lambda i: (i // 16, 0, 0)),
            pl.BlockSpec((1, 3, 427), lambda i: (i // 16, 0, 0)),
            pl.BlockSpec((1, 3, 427), lambda i: (i // 16, 0, 0)),
        ],
        out_specs=pl.BlockSpec((1, 48, 427), lambda i: (i, 0, 0)),
        out_shape=jax.ShapeDtypeStruct((32, 48, 427), F32),
    )(emb, ws, bs, wi)


def _inter_kernel(p_ref, h_ref, w_ref, b_ref, o_ref):
    p = p_ref[0]
    h = h_ref[0]
    p3 = jnp.broadcast_to(p[:, None, :], (48, 48, 427)).reshape(2304, 427)
    h3 = jnp.broadcast_to(h[None, :, :], (48, 48, 427)).reshape(2304, 427)
    inter = p3 * h3
    o_ref[0] = jnp.dot(inter, w_ref[...], preferred_element_type=F32, precision=_PHI) + b_ref[...]


def _inter_call(penc, henc, w, b):
    return _pcall(
        _inter_kernel,
        grid=(16,),
        in_specs=[
            pl.BlockSpec((1, 48, 427), lambda i: (i, 0, 0)),
            pl.BlockSpec((1, 48, 427), lambda i: (i, 0, 0)),
            pl.BlockSpec((427, 128), lambda i: (0, 0)),
            pl.BlockSpec((1, 128), lambda i: (0, 0)),
        ],
        out_specs=pl.BlockSpec((1, 2304, 128), lambda i: (i, 0, 0)),
        out_shape=jax.ShapeDtypeStruct((16, 2304, 128), F32),
    )(penc, henc, w, b)


def _sel_np(g):
    h = g // 2
    s = np.zeros((h * h, g * g), np.float32)
    r = np.arange(h * h)
    s[r, (2 * (r // h)) * g + 2 * (r % h)] = 1.0
    return s


_SEL = {g: _sel_np(g) for g in (48, 24, 12)}


def _block_kernel(x_ref, wc_ref, wt_ref, s_ref, o_ref, x_s, *, g, c0):
    gg = g * g
    x_s[:, :c0] = x_ref[0]
    jm = lax.broadcasted_iota(jnp.int32, (gg, 20), 0) % g
    mask_l = jm >= 1
    mask_r = jm <= g - 2
    zpad = jnp.zeros((g + 1, 180), F32)
    c = c0
    off = 0
    for _l in range(8):
        xv = x_s[:, :c]
        z = jnp.dot(xv, wc_ref[off : off + c, :], preferred_element_type=F32, precision=_PHI)
        zp = jnp.concatenate([zpad, z, zpad], axis=0)
        acc = jnp.zeros((gg, 20), F32)
        t = 0
        for a in (0, 1, 2):
            for b in (0, 1, 2):
                sh = (a - 1) * g + (b - 1)
                sl = zp[(g + 1) + sh : (g + 1) + sh + gg, 20 * t : 20 * t + 20]
                if b == 0:
                    sl = jnp.where(mask_l, sl, 0.0)
                elif b == 2:
                    sl = jnp.where(mask_r, sl, 0.0)
                acc = acc + sl
                t += 1
        x_s[:, c : c + 20] = jnp.maximum(acc, 0.0)
        off += c
        c += 20
    tr = jnp.maximum(jnp.dot(x_s[...], wt_ref[...], preferred_element_type=F32, precision=_PHI), 0.0)
    co = tr.shape[1]
    sh1 = jnp.concatenate([tr[1:], jnp.zeros((1, co), F32)], axis=0)
    m1 = jnp.maximum(tr, sh1)
    shg = jnp.concatenate([m1[g:], jnp.zeros((g, co), F32)], axis=0)
    m2 = jnp.maximum(m1, shg)
    o_ref[0] = jnp.dot(s_ref[...], m2, preferred_element_type=F32, precision=_PHI)


def _block_call(x, dn, b, g, c0):
    gg = g * g
    cfin = c0 + 160
    co = cfin // 2
    wcat = jnp.concatenate(
        [
            jnp.transpose(dn["b%d_l%d" % (b, l)], (2, 0, 1, 3)).reshape(c0 + 20 * l, 180)
            for l in range(8)
        ],
        axis=0,
    )
    sumc = wcat.shape[0]
    wt = dn["t%d" % b].reshape(cfin, co)
    sel = jnp.asarray(_SEL[g])
    return _pcall(
        functools.partial(_block_kernel, g=g, c0=c0),
        grid=(16,),
        in_specs=[
            pl.BlockSpec((1, gg, c0), lambda i: (i, 0, 0)),
            pl.BlockSpec((sumc, 180), lambda i: (0, 0)),
            pl.BlockSpec((cfin, co), lambda i: (0, 0)),
            pl.BlockSpec((gg // 4, gg), lambda i: (0, 0)),
        ],
        out_specs=pl.BlockSpec((1, gg // 4, co), lambda i: (i, 0, 0)),
        out_shape=jax.ShapeDtypeStruct((16, gg // 4, co), F32),
        scratch_shapes=[pltpu.VMEM((gg, cfin), F32)],
    )(x, wcat, wt, sel)


def _final_kernel(x_ref, w_ref, b_ref, o_ref):
    logits = jnp.dot(x_ref[...], w_ref[...], preferred_element_type=F32, precision=_PHI) + b_ref[...]
    logits = logits - jnp.max(logits, axis=1, keepdims=True)
    e = jnp.exp(logits)
    o_ref[...] = e / jnp.sum(e, axis=1, keepdims=True)


def _final_call(x, w, b):
    k = x.shape[1]
    return _pcall(
        _final_kernel,
        grid=(1,),
        in_specs=[
            pl.BlockSpec((16, k), lambda i: (0, 0)),
            pl.BlockSpec((k, 3), lambda i: (0, 0)),
            pl.BlockSpec((1, 3), lambda i: (0, 0)),
        ],
        out_specs=pl.BlockSpec((16, 3), lambda i: (0, 0)),
        out_shape=jax.ShapeDtypeStruct((16, 3), F32),
    )(x, w, b)


def kernel(prem_word, prem_char, prem_syn, hyp_word, hyp_char, hyp_syn, params):
    p = params
    idx = jnp.concatenate(
        [prem_word.reshape(-1), hyp_word.reshape(-1)]
    ).astype(jnp.int32)
    table4 = p["word_table"].reshape(10000, 1200)
    rows4 = _sc_gather_impl(table4, idx // 4)
    wemb = _qsel_call(rows4, (idx % 4).reshape(-1, 1))

    cids = jnp.concatenate(
        [prem_char.reshape(-1, 14), hyp_char.reshape(-1, 14)]
    ).astype(jnp.int32)
    wtr = jnp.transpose(p["char_conv_w"], (1, 0, 2))
    w2p = jnp.pad(wtr, ((0, 0), (0, 0), (0, 51))).reshape(30, 640)
    chf = _char_call(cids, p["char_table"], w2p, p["char_conv_b"].reshape(1, 77))

    syn = jnp.concatenate(
        [prem_syn.reshape(-1, 50), hyp_syn.reshape(-1, 50)]
    ).astype(F32)
    emb = jnp.concatenate([wemb, chf, syn], axis=1).reshape(32, 48, 427)

    encs = (p["enc_p"], p["enc_h"])
    ws = jnp.stack([jnp.concatenate([e["W1"], e["W2"], e["W3"]], axis=0) for e in encs])
    bs = jnp.stack([jnp.stack([e["b1"], e["b2"], e["b3"]]) for e in encs])
    wi = jnp.stack([e["w_itr"].reshape(3, 427) for e in encs])
    enc = _enc_call(emb, ws, bs, wi)

    x = _inter_call(
        enc[:16], enc[16:], p["fsd_w"].reshape(427, 128), p["fsd_b"].reshape(1, 128)
    )

    dn = p["densenet"]
    x = _block_call(x, dn, 0, 48, 128)
    x = _block_call(x, dn, 1, 24, 144)
    x = _block_call(x, dn, 2, 12, 152)

    xf = x.reshape(16, 36 * 156)
    return _final_call(xf, p["out_w"], p["out_b"].reshape(1, 3))


# speedup vs baseline: 1.8011x; 1.0020x over previous
import functools

import jax
import jax.numpy as jnp
import numpy as np
from jax import lax
from jax.experimental import pallas as pl
from jax.experimental.pallas import tpu as pltpu

_pcall = pl.pallas_call
_PHI = lax.Precision.DEFAULT

F32 = jnp.float32


def _sc_gather_impl(table, idx):
    from jax.experimental.pallas import tpu_sc as plsc

    info = plsc.get_sparse_core_info()
    nc, ns = info.num_cores, info.num_subcores
    nw = nc * ns
    b = idx.shape[0]
    d = table.shape[1]
    bpw = b // nw
    mesh = plsc.VectorSubcoreMesh(core_axis_name="c", subcore_axis_name="s")

    @functools.partial(
        pl.kernel,
        mesh=mesh,
        compiler_params=pltpu.CompilerParams(use_tc_tiling_on_sc=False),
        out_type=jax.ShapeDtypeStruct((b, d), F32),
        scratch_types=[
            pltpu.VMEM((bpw,), jnp.int32),
            pltpu.VMEM((bpw, d), F32),
            pltpu.SemaphoreType.DMA,
        ],
    )
    def gk(table_hbm, idx_hbm, out_hbm, idx_v, rows_v, sem):
        wid = lax.axis_index("s") * nc + lax.axis_index("c")
        base = wid * bpw
        pltpu.sync_copy(idx_hbm.at[pl.ds(base, bpw)], idx_v)
        pltpu.async_copy(table_hbm.at[idx_v], rows_v, sem).wait()
        pltpu.sync_copy(rows_v, out_hbm.at[pl.ds(base, bpw)])

    return gk(table, idx)


def _qsel_kernel(x_ref, k_ref, o_ref):
    x = x_ref[...]
    k = k_ref[...]
    s = [x[:, 300 * q : 300 * q + 300] for q in range(4)]
    o_ref[...] = jnp.where(
        k == 0, s[0], jnp.where(k == 1, s[1], jnp.where(k == 2, s[2], s[3]))
    )


def _qsel_call(rows4, kvec):
    n = rows4.shape[0]
    blk = 256
    return _pcall(
        _qsel_kernel,
        grid=(n // blk,),
        in_specs=[
            pl.BlockSpec((blk, 1200), lambda i: (i, 0)),
            pl.BlockSpec((blk, 1), lambda i: (i, 0)),
        ],
        out_specs=pl.BlockSpec((blk, 300), lambda i: (i, 0)),
        out_shape=jax.ShapeDtypeStruct((n, 300), F32),
    )(rows4, kvec)


def _char_kernel(ids_ref, t_ref, w_ref, b_ref, o_ref):
    n = ids_ref.shape[0]
    m = jnp.dot(t_ref[...], w_ref[...], preferred_element_type=F32, precision=_PHI)
    ids = ids_ref[...]
    iot = lax.broadcasted_iota(jnp.int32, (n, 128), 1)
    zs = []
    for c in range(14):
        oh = (ids[:, c : c + 1] == iot).astype(F32)
        zs.append(jnp.dot(oh, m, preferred_element_type=F32, precision=_PHI))
    best = None
    for pos in range(10):
        y = zs[pos][:, 0:128]
        for k in range(1, 5):
            y = y + zs[pos + k][:, 128 * k : 128 * k + 128]
        best = y if best is None else jnp.maximum(best, y)
    o_ref[...] = best[:, :77] + b_ref[...]


def _char_call(cids, ctable, w2p, cbias):
    nw = cids.shape[0]
    blk = 256
    return _pcall(
        _char_kernel,
        grid=(nw // blk,),
        in_specs=[
            pl.BlockSpec((blk, 14), lambda i: (i, 0)),
            pl.BlockSpec((128, 30), lambda i: (0, 0)),
            pl.BlockSpec((30, 640), lambda i: (0, 0)),
            pl.BlockSpec((1, 77), lambda i: (0, 0)),
        ],
        out_specs=pl.BlockSpec((blk, 77), lambda i: (i, 0)),
        out_shape=jax.ShapeDtypeStruct((nw, 77), F32),
    )(cids, ctable, w2p, cbias)


def _enc_kernel(emb_ref, ws_ref, bs_ref, wi_ref, o_ref):
    P = emb_ref[0]
    wa = wi_ref[0, 0:1, :]
    wb = wi_ref[0, 1:2, :]
    wc = wi_ref[0, 2:3, :]
    pa = jnp.sum(P * wa, axis=1, keepdims=True)
    dn = (((1,), (1,)), ((), ()))
    pbt = lax.dot_general(wb, P, dn, preferred_element_type=F32, precision=_PHI)
    cc = lax.dot_general(P * wc, P, dn, preferred_element_type=F32, precision=_PHI)
    a = pa + pbt + cc
    a = a - jnp.max(a, axis=1, keepdims=True)
    e = jnp.exp(a)
    att = e / jnp.sum(e, axis=1, keepdims=True)
    itr = jnp.dot(att, P, preferred_element_type=F32, precision=_PHI)
    cat = jnp.concatenate([P, itr], axis=1)
    ws = ws_ref[0]
    z = jnp.tanh(jnp.dot(cat, ws[0:854], preferred_element_type=F32, precision=_PHI) + bs_ref[0, 0:1, :])
    r = jax.nn.sigmoid(jnp.dot(cat, ws[854:1708], preferred_element_type=F32, precision=_PHI) + bs_ref[0, 1:2, :])
    f = jax.nn.sigmoid(jnp.dot(cat, ws[1708:2562], preferred_element_type=F32, precision=_PHI) + bs_ref[0, 2:3, :])
    o_ref[0] = r * P + f * z


def _enc_call(emb, ws, bs, wi):
    return _pcall(
        _enc_kernel,
        grid=(32,),
        in_specs=[
            pl.BlockSpec((1, 48, 427), lambda i: (i, 0, 0)),
            pl.BlockSpec((1, 2562, 427), lambda i: (i // 16, 0, 0)),
            pl.BlockSpec((1, 3, 427), lambda i: (i // 16, 0, 0)),
            pl.BlockSpec((1, 3, 427), lambda i: (i // 16, 0, 0)),
        ],
        out_specs=pl.BlockSpec((1, 48, 427), lambda i: (i, 0, 0)),
        out_shape=jax.ShapeDtypeStruct((32, 48, 427), F32),
    )(emb, ws, bs, wi)


def _inter_kernel(p_ref, h_ref, w_ref, b_ref, o_ref):
    p = p_ref[0]
    h = h_ref[0]
    p3 = jnp.broadcast_to(p[:, None, :], (48, 48, 427)).reshape(2304, 427)
    h3 = jnp.broadcast_to(h[None, :, :], (48, 48, 427)).reshape(2304, 427)
    inter = p3 * h3
    o_ref[0] = jnp.dot(inter, w_ref[...], preferred_element_type=F32, precision=_PHI) + b_ref[...]


def _inter_call(penc, henc, w, b):
    return _pcall(
        _inter_kernel,
        grid=(16,),
        in_specs=[
            pl.BlockSpec((1, 48, 427), lambda i: (i, 0, 0)),
            pl.BlockSpec((1, 48, 427), lambda i: (i, 0, 0)),
            pl.BlockSpec((427, 128), lambda i: (0, 0)),
            pl.BlockSpec((1, 128), lambda i: (0, 0)),
        ],
        out_specs=pl.BlockSpec((1, 2304, 128), lambda i: (i, 0, 0)),
        out_shape=jax.ShapeDtypeStruct((16, 2304, 128), F32),
    )(penc, henc, w, b)


def _sel_np(g):
    h = g // 2
    s = np.zeros((h * h, g * g), np.float32)
    r = np.arange(h * h)
    s[r, (2 * (r // h)) * g + 2 * (r % h)] = 1.0
    return s


_SEL = {g: _sel_np(g) for g in (48, 24, 12)}


def _block_kernel(x_ref, wc_ref, wt_ref, s_ref, o_ref, x_s, *, g, c0):
    gg = g * g
    x_s[:, :c0] = x_ref[0]
    jm = lax.broadcasted_iota(jnp.int32, (gg, 20), 0) % g
    mask_l = jm >= 1
    mask_r = jm <= g - 2
    zpad = jnp.zeros((g + 1, 180), F32)
    c = c0
    off = 0
    for _l in range(8):
        xv = x_s[:, :c].astype(jnp.bfloat16)
        z = jnp.dot(xv, wc_ref[off : off + c, :].astype(jnp.bfloat16),
                    preferred_element_type=F32)
        zp = jnp.concatenate([zpad, z, zpad], axis=0)
        acc = jnp.zeros((gg, 20), F32)
        t = 0
        for a in (0, 1, 2):
            for b in (0, 1, 2):
                sh = (a - 1) * g + (b - 1)
                sl = zp[(g + 1) + sh : (g + 1) + sh + gg, 20 * t : 20 * t + 20]
                if b == 0:
                    sl = jnp.where(mask_l, sl, 0.0)
                elif b == 2:
                    sl = jnp.where(mask_r, sl, 0.0)
                acc = acc + sl
                t += 1
        x_s[:, c : c + 20] = jnp.maximum(acc, 0.0)
        off += c
        c += 20
    tr = jnp.maximum(
        jnp.dot(x_s[...].astype(jnp.bfloat16), wt_ref[...].astype(jnp.bfloat16),
                preferred_element_type=F32), 0.0)
    co = tr.shape[1]
    sh1 = jnp.concatenate([tr[1:], jnp.zeros((1, co), F32)], axis=0)
    m1 = jnp.maximum(tr, sh1)
    shg = jnp.concatenate([m1[g:], jnp.zeros((g, co), F32)], axis=0)
    m2 = jnp.maximum(m1, shg)
    o_ref[0] = jnp.dot(s_ref[...], m2, preferred_element_type=F32, precision=_PHI)


def _block_call(x, dn, b, g, c0):
    gg = g * g
    cfin = c0 + 160
    co = cfin // 2
    wcat = jnp.concatenate(
        [
            jnp.transpose(dn["b%d_l%d" % (b, l)], (2, 0, 1, 3)).reshape(c0 + 20 * l, 180)
            for l in range(8)
        ],
        axis=0,
    )
    sumc = wcat.shape[0]
    wt = dn["t%d" % b].reshape(cfin, co)
    sel = jnp.asarray(_SEL[g])
    return _pcall(
        functools.partial(_block_kernel, g=g, c0=c0),
        grid=(16,),
        in_specs=[
            pl.BlockSpec((1, gg, c0), lambda i: (i, 0, 0)),
            pl.BlockSpec((sumc, 180), lambda i: (0, 0)),
            pl.BlockSpec((cfin, co), lambda i: (0, 0)),
            pl.BlockSpec((gg // 4, gg), lambda i: (0, 0)),
        ],
        out_specs=pl.BlockSpec((1, gg // 4, co), lambda i: (i, 0, 0)),
        out_shape=jax.ShapeDtypeStruct((16, gg // 4, co), F32),
        scratch_shapes=[pltpu.VMEM((gg, cfin), F32)],
    )(x, wcat, wt, sel)


def _final_kernel(x_ref, w_ref, b_ref, o_ref):
    logits = jnp.dot(x_ref[...], w_ref[...], preferred_element_type=F32, precision=_PHI) + b_ref[...]
    logits = logits - jnp.max(logits, axis=1, keepdims=True)
    e = jnp.exp(logits)
    o_ref[...] = e / jnp.sum(e, axis=1, keepdims=True)


def _final_call(x, w, b):
    k = x.shape[1]
    return _pcall(
        _final_kernel,
        grid=(1,),
        in_specs=[
            pl.BlockSpec((16, k), lambda i: (0, 0)),
            pl.BlockSpec((k, 3), lambda i: (0, 0)),
            pl.BlockSpec((1, 3), lambda i: (0, 0)),
        ],
        out_specs=pl.BlockSpec((16, 3), lambda i: (0, 0)),
        out_shape=jax.ShapeDtypeStruct((16, 3), F32),
    )(x, w, b)


def kernel(prem_word, prem_char, prem_syn, hyp_word, hyp_char, hyp_syn, params):
    p = params
    idx = jnp.concatenate(
        [prem_word.reshape(-1), hyp_word.reshape(-1)]
    ).astype(jnp.int32)
    table4 = p["word_table"].reshape(10000, 1200)
    rows4 = _sc_gather_impl(table4, idx // 4)
    wemb = _qsel_call(rows4, (idx % 4).reshape(-1, 1))

    cids = jnp.concatenate(
        [prem_char.reshape(-1, 14), hyp_char.reshape(-1, 14)]
    ).astype(jnp.int32)
    wtr = jnp.transpose(p["char_conv_w"], (1, 0, 2))
    w2p = jnp.pad(wtr, ((0, 0), (0, 0), (0, 51))).reshape(30, 640)
    chf = _char_call(cids, p["char_table"], w2p, p["char_conv_b"].reshape(1, 77))

    syn = jnp.concatenate(
        [prem_syn.reshape(-1, 50), hyp_syn.reshape(-1, 50)]
    ).astype(F32)
    emb = jnp.concatenate([wemb, chf, syn], axis=1).reshape(32, 48, 427)

    encs = (p["enc_p"], p["enc_h"])
    ws = jnp.stack([jnp.concatenate([e["W1"], e["W2"], e["W3"]], axis=0) for e in encs])
    bs = jnp.stack([jnp.stack([e["b1"], e["b2"], e["b3"]]) for e in encs])
    wi = jnp.stack([e["w_itr"].reshape(3, 427) for e in encs])
    enc = _enc_call(emb, ws, bs, wi)

    x = _inter_call(
        enc[:16], enc[16:], p["fsd_w"].reshape(427, 128), p["fsd_b"].reshape(1, 128)
    )

    dn = p["densenet"]
    x = _block_call(x, dn, 0, 48, 128)
    x = _block_call(x, dn, 1, 24, 144)
    x = _block_call(x, dn, 2, 12, 152)

    xf = x.reshape(16, 36 * 156)
    return _final_call(xf, p["out_w"], p["out_b"].reshape(1, 3))
